# baseline (speedup 1.0000x reference)
import jax
import jax.numpy as jnp
from jax import lax
from jax.experimental import pallas as pl
from jax.experimental.pallas import tpu as pltpu
from jax.experimental.pallas import tpu_sc as plsc

_CONTEXT = 8192
_DIM = 64
_VOCAB = 1000000
_LANES = 16
_NUM_WORKERS = 32
_BPW = _CONTEXT // _NUM_WORKERS


def _emb_body(x_hbm, t2_hbm, pos_hbm, out_hbm,
              xv, pv, gath, rows_v, posb, gsem, psem, xsem):
    wid = lax.axis_index("s") * 2 + lax.axis_index("c")
    base = wid * _BPW

    pltpu.async_copy(x_hbm.at[pl.ds(base, _BPW)], xv, xsem).wait()
    pos_cp = pltpu.async_copy(pos_hbm.at[pl.ds(base, _BPW), :], posb, psem)

    def pair_step(i, carry):
        sl = pl.ds(i * _LANES, _LANES)
        pv[sl] = lax.shift_right_logical(xv[sl], 1)
        return carry

    lax.fori_loop(0, _BPW // _LANES, pair_step, 0)

    gcp = pltpu.async_copy(t2_hbm.at[pv], gath, gsem)
    gcp.wait()
    pos_cp.wait()

    def extract_step(i, carry):
        j0 = i * _LANES
        svec = lax.shift_left(
            lax.bitwise_and(xv[pl.ds(j0, _LANES)], 1), 6)
        for jo in range(_LANES):
            j = j0 + jo
            s = svec[jo]
            for cc in range(_DIM // _LANES):
                sl = pl.ds(cc * _LANES, _LANES)
                rows_v[j, sl] = gath[j, pl.ds(s + cc * _LANES, _LANES)] \
                    + posb[j, sl]
        return carry

    lax.fori_loop(0, _BPW // _LANES, extract_step, 0)

    pltpu.sync_copy(rows_v, out_hbm.at[pl.ds(base, _BPW), :])


def kernel(x, table, pos_enc):
    table2 = table.reshape(_VOCAB // 2, 2 * _DIM)
    mesh = plsc.VectorSubcoreMesh(core_axis_name="c", subcore_axis_name="s")
    f = pl.kernel(
        _emb_body,
        mesh=mesh,
        out_type=jax.ShapeDtypeStruct((_CONTEXT, _DIM), jnp.float32),
        scratch_types=[
            pltpu.VMEM((_BPW,), jnp.int32),
            pltpu.VMEM((_BPW,), jnp.int32),
            pltpu.VMEM((_BPW, 2 * _DIM), jnp.float32),
            pltpu.VMEM((_BPW, _DIM), jnp.float32),
            pltpu.VMEM((_BPW, _DIM), jnp.float32),
            pltpu.SemaphoreType.DMA,
            pltpu.SemaphoreType.DMA,
            pltpu.SemaphoreType.DMA,
        ],
    )
    return f(x, table2, pos_enc)


# speedup vs baseline: 1.6278x; 1.6278x over previous
import jax
import jax.numpy as jnp
from jax import lax
from jax.experimental import pallas as pl
from jax.experimental.pallas import tpu as pltpu
from jax.experimental.pallas import tpu_sc as plsc

_CONTEXT = 8192
_DIM = 64
_VOCAB = 1000000
_LANES = 16
_NUM_WORKERS = 32
_BPW = _CONTEXT // _NUM_WORKERS
_CHUNK = 16
_NCHUNKS = _BPW // _CHUNK


def _emb_body(x_hbm, t_hbm, pos_hbm, out_hbm,
              xv, gb0, gb1, rows_v, posb, gsem0, gsem1, psem, xsem):
    wid = lax.axis_index("s") * 2 + lax.axis_index("c")
    base = wid * _BPW

    t3 = t_hbm.reshape(_VOCAB // 8, 8, _DIM)

    pltpu.async_copy(x_hbm.at[pl.ds(base, _BPW)], xv, xsem).wait()
    pos_cp = pltpu.async_copy(pos_hbm.at[pl.ds(base, _BPW), :], posb, psem)

    gbufs = (gb0, gb1)
    gsems = (gsem0, gsem1)

    def fire(c, buf, sem):
        tvec = lax.shift_right_logical(xv[pl.ds(c * _CHUNK, _CHUNK)], 3)
        for jj in range(_CHUNK):
            pltpu.async_copy(t3.at[tvec[jj]], buf.at[jj], sem)

    def drain(buf, sem):
        pltpu.make_async_copy(t3.at[pl.ds(0, _CHUNK)], buf, sem).wait()

    fire(0, gbufs[0], gsems[0])
    for c in range(_NCHUNKS):
        if c + 1 < _NCHUNKS:
            fire(c + 1, gbufs[(c + 1) % 2], gsems[(c + 1) % 2])
        drain(gbufs[c % 2], gsems[c % 2])
        if c == 0:
            pos_cp.wait()
        gath = gbufs[c % 2]
        svec = lax.bitwise_and(xv[pl.ds(c * _CHUNK, _CHUNK)], 7)
        for jo in range(_CHUNK):
            j = c * _CHUNK + jo
            s = svec[jo]
            for cc in range(_DIM // _LANES):
                sl = pl.ds(cc * _LANES, _LANES)
                rows_v[j, sl] = gath[jo, s, sl] + posb[j, sl]

    pltpu.sync_copy(rows_v, out_hbm.at[pl.ds(base, _BPW), :])


def kernel(x, table, pos_enc):
    mesh = plsc.VectorSubcoreMesh(core_axis_name="c", subcore_axis_name="s")
    f = pl.kernel(
        _emb_body,
        mesh=mesh,
        out_type=jax.ShapeDtypeStruct((_CONTEXT, _DIM), jnp.float32),
        scratch_types=[
            pltpu.VMEM((_BPW,), jnp.int32),
            pltpu.VMEM((_CHUNK, 8, _DIM), jnp.float32),
            pltpu.VMEM((_CHUNK, 8, _DIM), jnp.float32),
            pltpu.VMEM((_BPW, _DIM), jnp.float32),
            pltpu.VMEM((_BPW, _DIM), jnp.float32),
            pltpu.SemaphoreType.DMA,
            pltpu.SemaphoreType.DMA,
            pltpu.SemaphoreType.DMA,
            pltpu.SemaphoreType.DMA,
        ],
    )
    return f(x, table, pos_enc)


# speedup vs baseline: 2.3627x; 1.4515x over previous
import jax
import jax.numpy as jnp
from jax import lax
from jax.experimental import pallas as pl
from jax.experimental.pallas import tpu as pltpu
from jax.experimental.pallas import tpu_sc as plsc

_CONTEXT = 8192
_DIM = 64
_VOCAB = 1000000
_LANES = 16
_NUM_WORKERS = 32
_BPW = _CONTEXT // _NUM_WORKERS
_CHUNK = 16
_NCHUNKS = _BPW // _CHUNK


def _emb_body(x_hbm, t_hbm, pos_hbm, out_hbm,
              xv, gb0, gb1, rows_v, posb, gsem0, gsem1, psem, xsem):
    wid = lax.axis_index("s") * 2 + lax.axis_index("c")
    base = wid * _BPW

    t3 = t_hbm

    pltpu.async_copy(x_hbm.at[pl.ds(base, _BPW)], xv, xsem).wait()
    pos_cp = pltpu.async_copy(pos_hbm.at[pl.ds(base, _BPW), :], posb, psem)

    gbufs = (gb0, gb1)
    gsems = (gsem0, gsem1)

    def fire(c, buf, sem):
        tvec = lax.shift_right_logical(xv[pl.ds(c * _CHUNK, _CHUNK)], 3)
        for jj in range(_CHUNK):
            pltpu.async_copy(t3.at[tvec[jj]], buf.at[jj], sem)

    def drain(buf, sem):
        pltpu.make_async_copy(t3.at[pl.ds(0, _CHUNK)], buf, sem).wait()

    fire(0, gbufs[0], gsems[0])
    for c in range(_NCHUNKS):
        if c + 1 < _NCHUNKS:
            fire(c + 1, gbufs[(c + 1) % 2], gsems[(c + 1) % 2])
        drain(gbufs[c % 2], gsems[c % 2])
        if c == 0:
            pos_cp.wait()
        gath = gbufs[c % 2]
        svec = lax.bitwise_and(xv[pl.ds(c * _CHUNK, _CHUNK)], 7)
        for jo in range(_CHUNK):
            j = c * _CHUNK + jo
            s = svec[jo]
            for cc in range(_DIM // _LANES):
                sl = pl.ds(cc * _LANES, _LANES)
                rows_v[j, sl] = gath[jo, s, sl] + posb[j, sl]

    pltpu.sync_copy(rows_v, out_hbm.at[pl.ds(base, _BPW), :])


def kernel(x, table, pos_enc):
    table3 = table.reshape(_VOCAB // 8, 8, _DIM)
    mesh = plsc.VectorSubcoreMesh(core_axis_name="c", subcore_axis_name="s")
    f = pl.kernel(
        _emb_body,
        mesh=mesh,
        out_type=jax.ShapeDtypeStruct((_CONTEXT, _DIM), jnp.float32),
        scratch_types=[
            pltpu.VMEM((_BPW,), jnp.int32),
            pltpu.VMEM((_CHUNK, 8, _DIM), jnp.float32),
            pltpu.VMEM((_CHUNK, 8, _DIM), jnp.float32),
            pltpu.VMEM((_BPW, _DIM), jnp.float32),
            pltpu.VMEM((_BPW, _DIM), jnp.float32),
            pltpu.SemaphoreType.DMA,
            pltpu.SemaphoreType.DMA,
            pltpu.SemaphoreType.DMA,
            pltpu.SemaphoreType.DMA,
        ],
    )
    return f(x, table3, pos_enc)
